# baseline (speedup 1.0000x reference)
import dataclasses
import functools

import jax
import jax.numpy as jnp
from jax import lax
from jax.experimental import pallas as pl
from jax.experimental.pallas import tpu as pltpu
from jax.experimental.pallas import tpu_sc as plsc

B = 16384
H = 50
HP = 64
DIM = 32
L = 16

NC = 2
NS = 16
NW = NC * NS

BPW = B // NW
CH = 32
NCHUNK = BPW // CH
IDX_PER_CHUNK = CH * H
GFULL = IDX_PER_CHUNK // 128
GREM = IDX_PER_CHUNK - GFULL * 128

NROWS = 1000001
W = 512
NG = (NROWS - NROWS % 128) // W
TAIL = NROWS - NG * W
NPAD = NG * W + W
KMAX = NG // NW + 1

_mesh = plsc.VectorSubcoreMesh(core_axis_name="c", subcore_axis_name="s")


def _params(tc_tiling):
    cp = pltpu.CompilerParams()
    if "needs_layout_passes" in pltpu.CompilerParams.__dataclass_fields__:
        cp = dataclasses.replace(cp, needs_layout_passes=False)
    if "use_tc_tiling_on_sc" in pltpu.CompilerParams.__dataclass_fields__:
        cp = dataclasses.replace(cp, use_tc_tiling_on_sc=tc_tiling)
    return cp


@functools.partial(
    pl.kernel,
    mesh=_mesh,
    compiler_params=_params(True),
    out_type=jax.ShapeDtypeStruct((NPAD * DIM,), jnp.float32),
    scratch_types=[
        pltpu.VMEM((2, 4, 8, W), jnp.float32),
        pltpu.VMEM((2, W * DIM), jnp.float32),
        pltpu.SemaphoreType.DMA,
        pltpu.SemaphoreType.DMA,
        pltpu.SemaphoreType.DMA,
        pltpu.SemaphoreType.DMA,
    ],
)
def _repack_sc(tabt_hbm, tail_hbm, out_hbm, blk_v, out_v, isem0, isem1, osem0, osem1):
    wid = lax.axis_index("s") * NC + lax.axis_index("c")
    di = lax.iota(jnp.int32, L)
    t0 = di // 8
    s0 = di % 8
    t1 = t0 + 2
    isems = (isem0, isem1)
    osems = (osem0, osem1)

    def in_cp(k, b):
        g = k * NW + wid
        return pltpu.make_async_copy(
            tabt_hbm.at[:, :, pl.ds(g * W, W)], blk_v.at[b], isems[b]
        )

    def out_cp(k, b):
        g = k * NW + wid
        return pltpu.make_async_copy(
            out_v.at[b], out_hbm.at[pl.ds(g * W * DIM, W * DIM)], osems[b]
        )

    def valid(k):
        return k * NW + wid < NG

    def compute(b, lo, hi, shift, unroll=4):
        assert (hi - lo) % unroll == 0

        @pl.loop(lo, hi, step=unroll)
        def _item(i):
            for u in range(unroll):
                lane = jnp.broadcast_to(i + u, (L,)).astype(jnp.int32)
                v0 = plsc.load_gather(blk_v.at[b], [t0, s0, lane])
                v1 = plsc.load_gather(blk_v.at[b], [t1, s0, lane])
                out_v[b, pl.ds((i + u - shift) * DIM, L)] = v0
                out_v[b, pl.ds((i + u - shift) * DIM + L, L)] = v1

    in_cp(0, 0).start()

    @pl.loop(0, KMAX // 2)
    def _k2(k2):
        for b in (0, 1):
            k = k2 * 2 + b

            @pl.when(valid(k + 1))
            def _():
                in_cp(k + 1, 1 - b).start()

            @pl.when(valid(k))
            def _():
                in_cp(k, b).wait()

            @pl.when((k >= 2) & valid(k - 2))
            def _():
                out_cp(k - 2, b).wait()

            @pl.when(valid(k))
            def _():
                compute(b, 0, W, 0)
                out_cp(k, b).start()

    kl = KMAX - 1
    if KMAX % 2 == 1:
        @pl.when(valid(kl))
        def _():
            in_cp(kl, 0).wait()

        @pl.when(valid(kl - 2))
        def _():
            out_cp(kl - 2, 0).wait()

        @pl.when(valid(kl))
        def _():
            compute(0, 0, W, 0)
            out_cp(kl, 0).start()

    @pl.when(valid(kl - 1))
    def _():
        out_cp(kl - 1, (kl - 1) % 2).wait()

    @pl.when(valid(kl))
    def _():
        out_cp(kl, kl % 2).wait()

    @pl.when(wid == (NG % NW))
    def _tail():
        pltpu.sync_copy(tail_hbm, blk_v.at[0, :, :, pl.ds(0, 128)])
        compute(0, 128 - TAIL, 128, 128 - TAIL, unroll=1)
        pltpu.sync_copy(
            out_v.at[0, pl.ds(0, 128 * DIM)],
            out_hbm.at[pl.ds(NG * W * DIM, 128 * DIM)],
        )


@functools.partial(
    pl.kernel,
    mesh=_mesh,
    compiler_params=_params(False),
    out_type=jax.ShapeDtypeStruct((B, DIM), jnp.float32),
    scratch_types=[
        pltpu.VMEM((2, IDX_PER_CHUNK), jnp.int32),
        pltpu.VMEM((2, CH * HP), jnp.int32),
        pltpu.VMEM((2, IDX_PER_CHUNK, DIM), jnp.float32),
        pltpu.VMEM((2, CH, DIM), jnp.float32),
        pltpu.SemaphoreType.DMA,
        pltpu.SemaphoreType.DMA,
        pltpu.SemaphoreType.DMA,
        pltpu.SemaphoreType.DMA,
    ],
)
def _item_mean_sc(
    hist_hbm, histp_hbm, itab_hbm, out_hbm,
    idx_v, idxp_v, rows_v, out_v, gsem0, gsem1, osem0, osem1,
):
    wid = lax.axis_index("s") * NC + lax.axis_index("c")
    base = wid * BPW
    gsems = (gsem0, gsem1)
    osems = (osem0, osem1)

    def gathers(c, b):
        cps = []
        for j in range(GFULL):
            cps.append(
                pltpu.make_async_copy(
                    itab_hbm.at[idx_v.at[b, pl.ds(j * 128, 128)]],
                    rows_v.at[b, pl.ds(j * 128, 128)],
                    gsems[b],
                )
            )
        cps.append(
            pltpu.make_async_copy(
                itab_hbm.at[idx_v.at[b, pl.ds(GFULL * 128, GREM)]],
                rows_v.at[b, pl.ds(GFULL * 128, GREM)],
                gsems[b],
            )
        )
        return cps

    def stage_and_fire(c, b):
        rbase = base + c * CH
        pltpu.sync_copy(hist_hbm.at[pl.ds(rbase * H, IDX_PER_CHUNK)], idx_v.at[b])
        pltpu.sync_copy(histp_hbm.at[pl.ds(rbase * HP, CH * HP)], idxp_v.at[b])
        for cp in gathers(c, b):
            cp.start()

    def out_cp(c, b):
        rbase = base + c * CH
        return pltpu.make_async_copy(
            out_v.at[b], out_hbm.at[pl.ds(rbase, CH)], osems[b]
        )

    def compute(b):
        @pl.loop(0, CH)
        def _row(r):
            mcnt = jnp.zeros((L,), jnp.float32)
            for j in range(HP // L):
                v = idxp_v[b, pl.ds(r * HP + j * L, L)]
                mcnt = mcnt + jnp.where(v > 0, 1.0, 0.0).astype(jnp.float32)
            denom = jnp.broadcast_to(jnp.sum(mcnt), (L,)) + 1e-8
            recip = jnp.full((L,), 1.0, jnp.float32) / denom

            a0 = jnp.zeros((L,), jnp.float32)
            a1 = jnp.zeros((L,), jnp.float32)
            for h in range(H):
                a0 = a0 + rows_v[b, r * H + h, pl.ds(0, L)]
                a1 = a1 + rows_v[b, r * H + h, pl.ds(L, L)]

            out_v[b, r, pl.ds(0, L)] = a0 * recip
            out_v[b, r, pl.ds(L, L)] = a1 * recip

    stage_and_fire(0, 0)

    @pl.loop(0, NCHUNK // 2)
    def _c2(c2):
        for b in (0, 1):
            c = c2 * 2 + b

            @pl.when(c + 1 < NCHUNK)
            def _():
                stage_and_fire(c + 1, 1 - b)

            for cp in gathers(c, b):
                cp.wait()

            @pl.when(c >= 2)
            def _():
                out_cp(c - 2, b).wait()

            compute(b)
            out_cp(c, b).start()

    out_cp(NCHUNK - 2, 0).wait()
    out_cp(NCHUNK - 1, 1).wait()


@functools.partial(
    pl.kernel,
    mesh=_mesh,
    compiler_params=_params(False),
    out_type=jax.ShapeDtypeStruct((B, DIM), jnp.float32),
    scratch_types=[
        pltpu.VMEM((BPW,), jnp.int32),
        pltpu.VMEM((BPW, DIM), jnp.float32),
        pltpu.SemaphoreType.DMA,
    ],
)
def _user_gather_sc(uid_hbm, utab_hbm, out_hbm, uidx_v, urows_v, usem):
    wid = lax.axis_index("s") * NC + lax.axis_index("c")
    base = wid * BPW
    pltpu.sync_copy(uid_hbm.at[pl.ds(base, BPW)], uidx_v)
    copies = []
    for j in range(BPW // 128):
        copies.append(
            pltpu.async_copy(
                utab_hbm.at[uidx_v.at[pl.ds(j * 128, 128)]],
                urows_v.at[pl.ds(j * 128, 128)],
                usem,
            )
        )
    for cp in copies:
        cp.wait()
    pltpu.sync_copy(urows_v, out_hbm.at[pl.ds(base, BPW)])


def _combine_body(a_ref, b_ref, o_ref):
    o_ref[...] = a_ref[...] + b_ref[...]


_combine_tc = pl.pallas_call(
    _combine_body,
    out_shape=jax.ShapeDtypeStruct((B, DIM), jnp.float32),
    grid=(8,),
    in_specs=[
        pl.BlockSpec((B // 8, DIM), lambda i: (i, 0)),
        pl.BlockSpec((B // 8, DIM), lambda i: (i, 0)),
    ],
    out_specs=pl.BlockSpec((B // 8, DIM), lambda i: (i, 0)),
)


def _repack(table):
    tabt = table.T.reshape(4, 8, NROWS)
    tail = lax.slice(tabt, (0, 0, NROWS - 128), (4, 8, NROWS))
    return _repack_sc(tabt, tail).reshape(NPAD, DIM)


def kernel(user_ids, history, user_table, item_table):
    user_ids = user_ids.astype(jnp.int32)
    history = history.astype(jnp.int32)
    hist_flat = history.reshape(-1)
    histp_flat = jnp.pad(history, ((0, 0), (0, HP - H))).reshape(-1)
    item_lin = _repack(item_table)
    user_lin = _repack(user_table)
    hist_mean = _item_mean_sc(hist_flat, histp_flat, item_lin)
    user_rows = _user_gather_sc(user_ids, user_lin)
    return _combine_tc(user_rows, hist_mean)


# speedup vs baseline: 1.0042x; 1.0042x over previous
import dataclasses
import functools

import jax
import jax.numpy as jnp
from jax import lax
from jax.experimental import pallas as pl
from jax.experimental.pallas import tpu as pltpu
from jax.experimental.pallas import tpu_sc as plsc

B = 16384
H = 50
HP = 64
DIM = 32
L = 16

NC = 2
NS = 16
NW = NC * NS

BPW = B // NW
CH = 32
NCHUNK = BPW // CH
IDX_PER_CHUNK = CH * H
GFULL = IDX_PER_CHUNK // 128
GREM = IDX_PER_CHUNK - GFULL * 128

NROWS = 1000001
W = 256
NBUF = 6
NG = (NROWS - NROWS % 128) // W
TAIL = NROWS - NG * W
NPAD = NG * W + W
KMAX = NG // NW + 1

_mesh = plsc.VectorSubcoreMesh(core_axis_name="c", subcore_axis_name="s")


def _params(tc_tiling):
    cp = pltpu.CompilerParams()
    if "needs_layout_passes" in pltpu.CompilerParams.__dataclass_fields__:
        cp = dataclasses.replace(cp, needs_layout_passes=False)
    if "use_tc_tiling_on_sc" in pltpu.CompilerParams.__dataclass_fields__:
        cp = dataclasses.replace(cp, use_tc_tiling_on_sc=tc_tiling)
    return cp


@functools.partial(
    pl.kernel,
    mesh=_mesh,
    compiler_params=_params(True),
    out_type=jax.ShapeDtypeStruct((NPAD * DIM,), jnp.float32),
    scratch_types=[
        pltpu.VMEM((NBUF, 4, 8, W), jnp.float32),
        pltpu.VMEM((NBUF, W * DIM), jnp.float32),
    ]
    + [pltpu.SemaphoreType.DMA] * (2 * NBUF),
)
def _repack_sc(tabt_hbm, tail_hbm, out_hbm, blk_v, out_v, *sems):
    wid = lax.axis_index("s") * NC + lax.axis_index("c")
    di = lax.iota(jnp.int32, L)
    t0 = di // 8
    s0 = di % 8
    t1 = t0 + 2
    isems = sems[:NBUF]
    osems = sems[NBUF:]

    def in_cp(k, b):
        g = k * NW + wid
        return pltpu.make_async_copy(
            tabt_hbm.at[:, :, pl.ds(g * W, W)], blk_v.at[b], isems[b]
        )

    def out_cp(k, b):
        g = k * NW + wid
        return pltpu.make_async_copy(
            out_v.at[b], out_hbm.at[pl.ds(g * W * DIM, W * DIM)], osems[b]
        )

    def valid(k):
        return k * NW + wid < NG

    def compute(b, lo, hi, shift, unroll=4):
        assert (hi - lo) % unroll == 0

        @pl.loop(lo, hi, step=unroll)
        def _item(i):
            for u in range(unroll):
                lane = jnp.broadcast_to(i + u, (L,)).astype(jnp.int32)
                v0 = plsc.load_gather(blk_v.at[b], [t0, s0, lane])
                v1 = plsc.load_gather(blk_v.at[b], [t1, s0, lane])
                out_v[b, pl.ds((i + u - shift) * DIM, L)] = v0
                out_v[b, pl.ds((i + u - shift) * DIM + L, L)] = v1

    for j in range(NBUF - 1):
        @pl.when(valid(j))
        def _(j=j):
            in_cp(j, j % NBUF).start()

    @pl.loop(0, KMAX // NBUF)
    def _kN(kN):
        for b in range(NBUF):
            k = kN * NBUF + b
            nxt = k + NBUF - 1

            @pl.when(valid(nxt))
            def _():
                in_cp(nxt, (b + NBUF - 1) % NBUF).start()

            @pl.when(valid(k))
            def _():
                in_cp(k, b).wait()

            @pl.when((k >= NBUF) & valid(k - NBUF))
            def _():
                out_cp(k - NBUF, b).wait()

            @pl.when(valid(k))
            def _():
                compute(b, 0, W, 0)
                out_cp(k, b).start()

    for k0 in range((KMAX // NBUF) * NBUF, KMAX):
        b0 = k0 % NBUF
        nxt0 = k0 + NBUF - 1

        @pl.when(valid(nxt0))
        def _(k0=nxt0, b0=(b0 + NBUF - 1) % NBUF):
            in_cp(k0, b0).start()

        @pl.when(valid(k0))
        def _(k0=k0, b0=b0):
            in_cp(k0, b0).wait()

        @pl.when(valid(k0 - NBUF))
        def _(k0=k0, b0=b0):
            out_cp(k0 - NBUF, b0).wait()

        @pl.when(valid(k0))
        def _(k0=k0, b0=b0):
            compute(b0, 0, W, 0)
            out_cp(k0, b0).start()

    for k0 in range(max(0, KMAX - NBUF), KMAX):
        @pl.when(valid(k0))
        def _(k0=k0):
            out_cp(k0, k0 % NBUF).wait()

    @pl.when(wid == (NG % NW))
    def _tail():
        pltpu.sync_copy(tail_hbm, blk_v.at[0, :, :, pl.ds(0, 128)])
        compute(0, 128 - TAIL, 128, 128 - TAIL, unroll=1)
        pltpu.sync_copy(
            out_v.at[0, pl.ds(0, 128 * DIM)],
            out_hbm.at[pl.ds(NG * W * DIM, 128 * DIM)],
        )


@functools.partial(
    pl.kernel,
    mesh=_mesh,
    compiler_params=_params(False),
    out_type=jax.ShapeDtypeStruct((B, DIM), jnp.float32),
    scratch_types=[
        pltpu.VMEM((2, IDX_PER_CHUNK), jnp.int32),
        pltpu.VMEM((2, CH * HP), jnp.int32),
        pltpu.VMEM((2, IDX_PER_CHUNK, DIM), jnp.float32),
        pltpu.VMEM((2, CH, DIM), jnp.float32),
        pltpu.SemaphoreType.DMA,
        pltpu.SemaphoreType.DMA,
        pltpu.SemaphoreType.DMA,
        pltpu.SemaphoreType.DMA,
    ],
)
def _item_mean_sc(
    hist_hbm, histp_hbm, itab_hbm, out_hbm,
    idx_v, idxp_v, rows_v, out_v, gsem0, gsem1, osem0, osem1,
):
    wid = lax.axis_index("s") * NC + lax.axis_index("c")
    base = wid * BPW
    gsems = (gsem0, gsem1)
    osems = (osem0, osem1)

    def gathers(c, b):
        cps = []
        for j in range(GFULL):
            cps.append(
                pltpu.make_async_copy(
                    itab_hbm.at[idx_v.at[b, pl.ds(j * 128, 128)]],
                    rows_v.at[b, pl.ds(j * 128, 128)],
                    gsems[b],
                )
            )
        cps.append(
            pltpu.make_async_copy(
                itab_hbm.at[idx_v.at[b, pl.ds(GFULL * 128, GREM)]],
                rows_v.at[b, pl.ds(GFULL * 128, GREM)],
                gsems[b],
            )
        )
        return cps

    def stage_and_fire(c, b):
        rbase = base + c * CH
        pltpu.sync_copy(hist_hbm.at[pl.ds(rbase * H, IDX_PER_CHUNK)], idx_v.at[b])
        pltpu.sync_copy(histp_hbm.at[pl.ds(rbase * HP, CH * HP)], idxp_v.at[b])
        for cp in gathers(c, b):
            cp.start()

    def out_cp(c, b):
        rbase = base + c * CH
        return pltpu.make_async_copy(
            out_v.at[b], out_hbm.at[pl.ds(rbase, CH)], osems[b]
        )

    def compute(b):
        @pl.loop(0, CH)
        def _row(r):
            mcnt = jnp.zeros((L,), jnp.float32)
            for j in range(HP // L):
                v = idxp_v[b, pl.ds(r * HP + j * L, L)]
                mcnt = mcnt + jnp.where(v > 0, 1.0, 0.0).astype(jnp.float32)
            denom = jnp.broadcast_to(jnp.sum(mcnt), (L,)) + 1e-8
            recip = jnp.full((L,), 1.0, jnp.float32) / denom

            a0 = jnp.zeros((L,), jnp.float32)
            a1 = jnp.zeros((L,), jnp.float32)
            for h in range(H):
                a0 = a0 + rows_v[b, r * H + h, pl.ds(0, L)]
                a1 = a1 + rows_v[b, r * H + h, pl.ds(L, L)]

            out_v[b, r, pl.ds(0, L)] = a0 * recip
            out_v[b, r, pl.ds(L, L)] = a1 * recip

    stage_and_fire(0, 0)

    @pl.loop(0, NCHUNK // 2)
    def _c2(c2):
        for b in (0, 1):
            c = c2 * 2 + b

            @pl.when(c + 1 < NCHUNK)
            def _():
                stage_and_fire(c + 1, 1 - b)

            for cp in gathers(c, b):
                cp.wait()

            @pl.when(c >= 2)
            def _():
                out_cp(c - 2, b).wait()

            compute(b)
            out_cp(c, b).start()

    out_cp(NCHUNK - 2, 0).wait()
    out_cp(NCHUNK - 1, 1).wait()


@functools.partial(
    pl.kernel,
    mesh=_mesh,
    compiler_params=_params(False),
    out_type=jax.ShapeDtypeStruct((B, DIM), jnp.float32),
    scratch_types=[
        pltpu.VMEM((BPW,), jnp.int32),
        pltpu.VMEM((BPW, DIM), jnp.float32),
        pltpu.SemaphoreType.DMA,
    ],
)
def _user_gather_sc(uid_hbm, utab_hbm, out_hbm, uidx_v, urows_v, usem):
    wid = lax.axis_index("s") * NC + lax.axis_index("c")
    base = wid * BPW
    pltpu.sync_copy(uid_hbm.at[pl.ds(base, BPW)], uidx_v)
    copies = []
    for j in range(BPW // 128):
        copies.append(
            pltpu.async_copy(
                utab_hbm.at[uidx_v.at[pl.ds(j * 128, 128)]],
                urows_v.at[pl.ds(j * 128, 128)],
                usem,
            )
        )
    for cp in copies:
        cp.wait()
    pltpu.sync_copy(urows_v, out_hbm.at[pl.ds(base, BPW)])


def _combine_body(a_ref, b_ref, o_ref):
    o_ref[...] = a_ref[...] + b_ref[...]


_combine_tc = pl.pallas_call(
    _combine_body,
    out_shape=jax.ShapeDtypeStruct((B, DIM), jnp.float32),
    grid=(8,),
    in_specs=[
        pl.BlockSpec((B // 8, DIM), lambda i: (i, 0)),
        pl.BlockSpec((B // 8, DIM), lambda i: (i, 0)),
    ],
    out_specs=pl.BlockSpec((B // 8, DIM), lambda i: (i, 0)),
)


def _repack(table):
    tabt = table.T.reshape(4, 8, NROWS)
    tail = lax.slice(tabt, (0, 0, NROWS - 128), (4, 8, NROWS))
    return _repack_sc(tabt, tail).reshape(NPAD, DIM)


def kernel(user_ids, history, user_table, item_table):
    user_ids = user_ids.astype(jnp.int32)
    history = history.astype(jnp.int32)
    hist_flat = history.reshape(-1)
    histp_flat = jnp.pad(history, ((0, 0), (0, HP - H))).reshape(-1)
    item_lin = _repack(item_table)
    user_lin = _repack(user_table)
    hist_mean = _item_mean_sc(hist_flat, histp_flat, item_lin)
    user_rows = _user_gather_sc(user_ids, user_lin)
    return _combine_tc(user_rows, hist_mean)


# speedup vs baseline: 1.6457x; 1.6389x over previous
import dataclasses
import functools

import jax
import jax.numpy as jnp
from jax import lax
from jax.experimental import pallas as pl
from jax.experimental.pallas import tpu as pltpu
from jax.experimental.pallas import tpu_sc as plsc

B = 16384
H = 50
HP = 64
DIM = 32
L = 16

NC = 2
NS = 16
NW = NC * NS

BPW = B // NW
CH = 32
NCHUNK = BPW // CH
IDX_PER_CHUNK = CH * H
GFULL = IDX_PER_CHUNK // 128
GREM = IDX_PER_CHUNK - GFULL * 128

_mesh = plsc.VectorSubcoreMesh(core_axis_name="c", subcore_axis_name="s")


def _params(tc_tiling):
    cp = pltpu.CompilerParams()
    if "needs_layout_passes" in pltpu.CompilerParams.__dataclass_fields__:
        cp = dataclasses.replace(cp, needs_layout_passes=False)
    if "use_tc_tiling_on_sc" in pltpu.CompilerParams.__dataclass_fields__:
        cp = dataclasses.replace(cp, use_tc_tiling_on_sc=tc_tiling)
    return cp


@functools.partial(
    pl.kernel,
    mesh=_mesh,
    compiler_params=_params(False),
    out_type=jax.ShapeDtypeStruct((B, DIM), jnp.float32),
    scratch_types=[
        pltpu.VMEM((2, IDX_PER_CHUNK), jnp.int32),
        pltpu.VMEM((2, CH * HP), jnp.int32),
        pltpu.VMEM((2, IDX_PER_CHUNK, DIM), jnp.float32),
        pltpu.VMEM((2, CH, DIM), jnp.float32),
        pltpu.SemaphoreType.DMA,
        pltpu.SemaphoreType.DMA,
        pltpu.SemaphoreType.DMA,
        pltpu.SemaphoreType.DMA,
    ],
)
def _item_mean_sc(
    hist_hbm, histp_hbm, itab_hbm, out_hbm,
    idx_v, idxp_v, rows_v, out_v, gsem0, gsem1, osem0, osem1,
):
    wid = lax.axis_index("s") * NC + lax.axis_index("c")
    base = wid * BPW
    gsems = (gsem0, gsem1)
    osems = (osem0, osem1)

    def gathers(c, b):
        cps = []
        for j in range(GFULL):
            cps.append(
                pltpu.make_async_copy(
                    itab_hbm.at[idx_v.at[b, pl.ds(j * 128, 128)]],
                    rows_v.at[b, pl.ds(j * 128, 128)],
                    gsems[b],
                )
            )
        cps.append(
            pltpu.make_async_copy(
                itab_hbm.at[idx_v.at[b, pl.ds(GFULL * 128, GREM)]],
                rows_v.at[b, pl.ds(GFULL * 128, GREM)],
                gsems[b],
            )
        )
        return cps

    def stage_and_fire(c, b):
        rbase = base + c * CH
        pltpu.sync_copy(hist_hbm.at[pl.ds(rbase * H, IDX_PER_CHUNK)], idx_v.at[b])
        pltpu.sync_copy(histp_hbm.at[pl.ds(rbase * HP, CH * HP)], idxp_v.at[b])
        for cp in gathers(c, b):
            cp.start()

    def out_cp(c, b):
        rbase = base + c * CH
        return pltpu.make_async_copy(
            out_v.at[b], out_hbm.at[pl.ds(rbase, CH)], osems[b]
        )

    def compute(b):
        @pl.loop(0, CH)
        def _row(r):
            mcnt = jnp.zeros((L,), jnp.float32)
            for j in range(HP // L):
                v = idxp_v[b, pl.ds(r * HP + j * L, L)]
                mcnt = mcnt + jnp.where(v > 0, 1.0, 0.0).astype(jnp.float32)
            denom = jnp.broadcast_to(jnp.sum(mcnt), (L,)) + 1e-8
            recip = jnp.full((L,), 1.0, jnp.float32) / denom

            a0 = jnp.zeros((L,), jnp.float32)
            a1 = jnp.zeros((L,), jnp.float32)
            for h in range(H):
                a0 = a0 + rows_v[b, r * H + h, pl.ds(0, L)]
                a1 = a1 + rows_v[b, r * H + h, pl.ds(L, L)]

            out_v[b, r, pl.ds(0, L)] = a0 * recip
            out_v[b, r, pl.ds(L, L)] = a1 * recip

    stage_and_fire(0, 0)

    @pl.loop(0, NCHUNK // 2)
    def _c2(c2):
        for b in (0, 1):
            c = c2 * 2 + b

            @pl.when(c + 1 < NCHUNK)
            def _():
                stage_and_fire(c + 1, 1 - b)

            for cp in gathers(c, b):
                cp.wait()

            @pl.when(c >= 2)
            def _():
                out_cp(c - 2, b).wait()

            compute(b)
            out_cp(c, b).start()

    out_cp(NCHUNK - 2, 0).wait()
    out_cp(NCHUNK - 1, 1).wait()


@functools.partial(
    pl.kernel,
    mesh=_mesh,
    compiler_params=_params(False),
    out_type=jax.ShapeDtypeStruct((B, DIM), jnp.float32),
    scratch_types=[
        pltpu.VMEM((BPW,), jnp.int32),
        pltpu.VMEM((BPW, DIM), jnp.float32),
        pltpu.SemaphoreType.DMA,
    ],
)
def _user_gather_sc(uid_hbm, utab_hbm, out_hbm, uidx_v, urows_v, usem):
    wid = lax.axis_index("s") * NC + lax.axis_index("c")
    base = wid * BPW
    pltpu.sync_copy(uid_hbm.at[pl.ds(base, BPW)], uidx_v)
    copies = []
    for j in range(BPW // 128):
        copies.append(
            pltpu.async_copy(
                utab_hbm.at[uidx_v.at[pl.ds(j * 128, 128)]],
                urows_v.at[pl.ds(j * 128, 128)],
                usem,
            )
        )
    for cp in copies:
        cp.wait()
    pltpu.sync_copy(urows_v, out_hbm.at[pl.ds(base, BPW)])


def _combine_body(a_ref, b_ref, o_ref):
    o_ref[...] = a_ref[...] + b_ref[...]


_combine_tc = pl.pallas_call(
    _combine_body,
    out_shape=jax.ShapeDtypeStruct((B, DIM), jnp.float32),
    grid=(8,),
    in_specs=[
        pl.BlockSpec((B // 8, DIM), lambda i: (i, 0)),
        pl.BlockSpec((B // 8, DIM), lambda i: (i, 0)),
    ],
    out_specs=pl.BlockSpec((B // 8, DIM), lambda i: (i, 0)),
)


def kernel(user_ids, history, user_table, item_table):
    user_ids = user_ids.astype(jnp.int32)
    history = history.astype(jnp.int32)
    hist_flat = history.reshape(-1)
    histp_flat = jnp.pad(history, ((0, 0), (0, HP - H))).reshape(-1)
    hist_mean = _item_mean_sc(hist_flat, histp_flat, item_table)
    user_rows = _user_gather_sc(user_ids, user_table)
    return _combine_tc(user_rows, hist_mean)
